# baseline (speedup 1.0000x reference)
import jax
import jax.numpy as jnp
from jax.experimental import pallas as pl
from jax.experimental.pallas import tpu as pltpu

_N, _P, _B, _D, _M = 100000, 256, 4096, 128, 32
_SLOTS = 4
_CAP = 6400
_CHUNKS = [2400, 3200, 4800] + [6400] * 14
_STARTS = [sum(_CHUNKS[:j]) for j in range(len(_CHUNKS))]
assert sum(_CHUNKS) == _N and all(c % 8 == 0 for c in _CHUNKS)


def _body(meta_hbm, card_hbm, pools_hbm, w_ref, b_ref, out_ref,
          meta_buf, card_buf, pools_buf, acc_ref,
          meta_sems, card_sems, pool_sem):
    pltpu.make_async_copy(pools_hbm, pools_buf, pool_sem).start()

    def _copy(j, slot):
        st, sz = _STARTS[j], _CHUNKS[j]
        m = pltpu.make_async_copy(meta_hbm.at[pl.ds(st, sz), :],
                                  meta_buf.at[slot, pl.ds(0, sz), :],
                                  meta_sems.at[slot])
        c = pltpu.make_async_copy(card_hbm.at[pl.ds(st, sz), :],
                                  card_buf.at[slot, pl.ds(0, sz), :],
                                  card_sems.at[slot])
        return m, c

    for k in range(_SLOTS):
        m, c = _copy(k, k)
        m.start()
        c.start()

    acc_ref[...] = jnp.zeros_like(acc_ref)

    for j, sz in enumerate(_CHUNKS):
        slot = j % _SLOTS
        m, c = _copy(j, slot)
        m.wait()
        c.wait()
        card_blk = card_buf[slot, pl.ds(0, sz), :]
        pre = jnp.dot(card_blk, w_ref[...],
                      preferred_element_type=jnp.float32) + b_ref[...]
        transformed = pre * jax.nn.sigmoid(pre)
        acc_ref[...] += jax.lax.dot_general(
            meta_buf[slot, pl.ds(0, sz), :].astype(jnp.bfloat16),
            transformed.astype(jnp.bfloat16),
            (((0,), (0,)), ((), ())),
            preferred_element_type=jnp.float32)
        if j + _SLOTS < len(_CHUNKS):
            m2, c2 = _copy(j + _SLOTS, slot)
            m2.start()
            c2.start()

    pltpu.make_async_copy(pools_hbm, pools_buf, pool_sem).wait()
    out_ref[...] = jnp.dot(pools_buf[...], acc_ref[...],
                           preferred_element_type=jnp.float32)


def kernel(batch_pools, metapath, card_embeddings, W, b_dense):
    b2 = b_dense.reshape(1, _M)
    return pl.pallas_call(
        _body,
        in_specs=[
            pl.BlockSpec(memory_space=pl.ANY),
            pl.BlockSpec(memory_space=pl.ANY),
            pl.BlockSpec(memory_space=pl.ANY),
            pl.BlockSpec(memory_space=pltpu.VMEM),
            pl.BlockSpec(memory_space=pltpu.VMEM),
        ],
        out_specs=pl.BlockSpec(memory_space=pltpu.VMEM),
        out_shape=jax.ShapeDtypeStruct((_B, _M), jnp.float32),
        scratch_shapes=[
            pltpu.VMEM((_SLOTS, _CAP, _P), jnp.float32),
            pltpu.VMEM((_SLOTS, _CAP, _D), jnp.float32),
            pltpu.VMEM((_B, _P), jnp.float32),
            pltpu.VMEM((_P, _M), jnp.float32),
            pltpu.SemaphoreType.DMA((_SLOTS,)),
            pltpu.SemaphoreType.DMA((_SLOTS,)),
            pltpu.SemaphoreType.DMA,
        ],
    )(metapath, card_embeddings, batch_pools, W, b2)


# speedup vs baseline: 1.0750x; 1.0750x over previous
import jax
import jax.numpy as jnp
from jax.experimental import pallas as pl
from jax.experimental.pallas import tpu as pltpu

_N, _P, _B, _D, _M = 100000, 256, 4096, 128, 32
_BN = 10000
_G = _N // _BN


def _fused_body(meta_ref, card_ref, w_ref, b_ref, pools_hbm, out_ref,
                acc_ref, pools_buf, pool_sem):
    i = pl.program_id(0)

    @pl.when(i == 0)
    def _init():
        acc_ref[...] = jnp.zeros_like(acc_ref)
        pltpu.make_async_copy(pools_hbm, pools_buf, pool_sem).start()

    pre = jnp.dot(card_ref[...], w_ref[...],
                  preferred_element_type=jnp.float32) + b_ref[...]
    transformed = pre * jax.nn.sigmoid(pre)
    acc_ref[...] += jax.lax.dot_general(
        meta_ref[...].astype(jnp.bfloat16), transformed.astype(jnp.bfloat16),
        (((0,), (0,)), ((), ())),
        preferred_element_type=jnp.float32)

    @pl.when(i == _G - 1)
    def _finish():
        pltpu.make_async_copy(pools_hbm, pools_buf, pool_sem).wait()
        out_ref[...] = jnp.dot(pools_buf[...], acc_ref[...],
                               preferred_element_type=jnp.float32)


def kernel(batch_pools, metapath, card_embeddings, W, b_dense):
    b2 = b_dense.reshape(1, _M)
    return pl.pallas_call(
        _fused_body,
        grid=(_G,),
        in_specs=[
            pl.BlockSpec((_BN, _P), lambda i: (i, 0)),
            pl.BlockSpec((_BN, _D), lambda i: (i, 0)),
            pl.BlockSpec((_D, _M), lambda i: (0, 0)),
            pl.BlockSpec((1, _M), lambda i: (0, 0)),
            pl.BlockSpec(memory_space=pl.ANY),
        ],
        out_specs=pl.BlockSpec((_B, _M), lambda i: (0, 0)),
        out_shape=jax.ShapeDtypeStruct((_B, _M), jnp.float32),
        scratch_shapes=[
            pltpu.VMEM((_P, _M), jnp.float32),
            pltpu.VMEM((_B, _P), jnp.float32),
            pltpu.SemaphoreType.DMA,
        ],
    )(metapath, card_embeddings, W, b2, batch_pools)


# speedup vs baseline: 1.0792x; 1.0039x over previous
import jax
import jax.numpy as jnp
from jax.experimental import pallas as pl
from jax.experimental.pallas import tpu as pltpu

_N, _P, _B, _D, _M = 100000, 256, 4096, 128, 32
_BN = 10000
_G = _N // _BN


def _fused_body(meta_ref, card_ref, w_ref, b_ref, pools_ref, out_ref, acc_ref):
    i = pl.program_id(0)

    @pl.when(i == 0)
    def _init():
        acc_ref[...] = jnp.zeros_like(acc_ref)

    pre = jnp.dot(card_ref[...], w_ref[...],
                  preferred_element_type=jnp.float32) + b_ref[...]
    transformed = pre * jax.nn.sigmoid(pre)
    acc_ref[...] += jax.lax.dot_general(
        meta_ref[...].astype(jnp.bfloat16), transformed.astype(jnp.bfloat16),
        (((0,), (0,)), ((), ())),
        preferred_element_type=jnp.float32)

    @pl.when(i == _G - 1)
    def _finish():
        out_ref[...] = jnp.dot(pools_ref[...], acc_ref[...],
                               preferred_element_type=jnp.float32)


def kernel(batch_pools, metapath, card_embeddings, W, b_dense):
    b2 = b_dense.reshape(1, _M)
    return pl.pallas_call(
        _fused_body,
        grid=(_G,),
        in_specs=[
            pl.BlockSpec((_BN, _P), lambda i: (i, 0)),
            pl.BlockSpec((_BN, _D), lambda i: (i, 0)),
            pl.BlockSpec((_D, _M), lambda i: (0, 0)),
            pl.BlockSpec((1, _M), lambda i: (0, 0)),
            pl.BlockSpec((_B, _P), lambda i: (0, 0)),
        ],
        out_specs=pl.BlockSpec((_B, _M), lambda i: (0, 0)),
        out_shape=jax.ShapeDtypeStruct((_B, _M), jnp.float32),
        scratch_shapes=[pltpu.VMEM((_P, _M), jnp.float32)],
    )(metapath, card_embeddings, W, b2, batch_pools)
